# Initial kernel scaffold; baseline (speedup 1.0000x reference)
import jax
import jax.numpy as jnp
from jax.experimental import pallas as pl


def kernel(vertex_positions, vertex_features, edge_index, feat0, feat1, feat2, feat3, W_lin0, w0_g0, w1_g0, w0_g1, w1_g1, w0_g2, w1_g2, W_lin1):
    raise NotImplementedError("write your pallas kernel here")


# speedup vs baseline: 2.6959x; 2.6959x over previous
import functools

import jax
import jax.numpy as jnp
from jax import lax
from jax.experimental import pallas as pl
from jax.experimental.pallas import tpu as pltpu
from jax.experimental.pallas import tpu_sc as plsc

N = 10000
NPAD = 10240
E = 320000
D = 128
SIZES = (56, 28, 14, 7)
CHANS = (256, 512, 1024, 2048)
COFF = (0, 256, 768, 1792, 3840)
PPAD = (3136, 784, 200, 56)
OFFS = (0, 3136, 3920, 4120)
ZROW = 4176
TROWS = 4184

NC, NS = 2, 16
NW = NC * NS
VPW = NPAD // NW
VCH = 80
EPT = E // NS
ECH = 128
NFULL = EPT // ECH
ETAIL = EPT - NFULL * ECH
ROWS_PW = NPAD // NS
DH = D // 2

_PREC = jax.lax.Precision.HIGHEST
_F32 = jnp.float32


def _dot(a, b):
    return jnp.dot(a, b, preferred_element_type=_F32, precision=_PREC)


def _prep_body(post_ref, f0_ref, f1_ref, f2_ref, f3_ref, wt_ref, t_ref, idx_ref):
    t_ref[0:3136, :] = _dot(f0_ref[...], wt_ref[COFF[0]:COFF[1], :])
    t_ref[3136:3920, :] = _dot(f1_ref[...], wt_ref[COFF[1]:COFF[2], :])
    t_ref[3920:4120, :] = _dot(f2_ref[...], wt_ref[COFF[2]:COFF[3], :])
    t_ref[4120:4176, :] = _dot(f3_ref[...], wt_ref[COFF[3]:COFF[4], :])
    t_ref[4176:4184, :] = jnp.zeros((8, D), _F32)

    px = post_ref[0:1, :]
    py = post_ref[1:2, :]
    pz = post_ref[2:3, :]
    h = 248.0 * (py / pz) + 111.5
    w = 248.0 * (px / (-pz)) + 111.5
    h = jnp.clip(h, 0.0, 223.0)
    w = jnp.clip(w, 0.0, 223.0)
    for s in range(4):
        size = SIZES[s]
        inv = jnp.float32(size / 224.0)
        x = w * inv
        y = h * inv
        x1 = jnp.floor(x).astype(jnp.int32)
        y1 = jnp.floor(y).astype(jnp.int32)
        x2 = jnp.minimum(jnp.ceil(x), float(size - 1)).astype(jnp.int32)
        y2 = jnp.minimum(jnp.ceil(y), float(size - 1)).astype(jnp.int32)
        m = ((x2 - x1) * (y2 - y1)) == 1
        lin = OFFS[s] + x1 * size + y1
        lin = jnp.clip(lin, OFFS[s], OFFS[s] + size * size - 1)
        idx_ref[s:s + 1, :] = jnp.where(m, lin, ZROW)


def _prep(post, f0t, f1t, f2t, f3t, wt):
    return pl.pallas_call(
        _prep_body,
        out_shape=(
            jax.ShapeDtypeStruct((TROWS, D), _F32),
            jax.ShapeDtypeStruct((4, NPAD), jnp.int32),
        ),
    )(post, f0t, f1t, f2t, f3t, wt)


def _proj_body(t_hbm, idx_hbm, out_hbm,
               i0, i1, i2, i3, r0, r1, r2, r3, acc, sem):
    cid = lax.axis_index("c")
    sid = lax.axis_index("s")
    wid = sid * NC + cid
    ibufs = (i0, i1, i2, i3)
    rbufs = (r0, r1, r2, r3)

    def chunk(ch, carry):
        base = wid * VPW + ch * VCH
        for s in range(4):
            pltpu.sync_copy(idx_hbm.at[pl.ds(s * NPAD + base, VCH)], ibufs[s])
        ds = [pltpu.async_copy(t_hbm.at[ibufs[s]], rbufs[s], sem)
              for s in range(4)]
        for d in ds:
            d.wait()

        def row(i, c):
            for j in range(8):
                sl = pl.ds(j * 16, 16)
                acc[i, sl] = r0[i, sl] + r1[i, sl] + r2[i, sl] + r3[i, sl]
            return c

        lax.fori_loop(0, VCH, row, 0, unroll=False)
        pltpu.sync_copy(acc, out_hbm.at[pl.ds(base, VCH)])
        return carry

    lax.fori_loop(0, VPW // VCH, chunk, 0, unroll=False)


def _proj(t, idx_flat):
    mesh = plsc.VectorSubcoreMesh(core_axis_name="c", subcore_axis_name="s",
                                  num_cores=NC, num_subcores=NS)
    f = pl.kernel(
        _proj_body, mesh=mesh,
        out_type=jax.ShapeDtypeStruct((NPAD, D), _F32),
        scratch_types=[pltpu.VMEM((VCH,), jnp.int32)] * 4
        + [pltpu.VMEM((VCH, D), _F32)] * 5
        + [pltpu.SemaphoreType.DMA],
    )
    return f(t, idx_flat)


def _seg_body(src_hbm, dst_hbm, b_hbm, out_hbm,
              sidx, didx, rows, sidx_t, didx_t, rows_t, stage, acc_sh, sem):
    cid = lax.axis_index("c")
    sid = lax.axis_index("s")
    ebase = sid * EPT
    roff = cid * NPAD

    def zrow(i, c):
        for j in range(4):
            stage[i, pl.ds(j * 16, 16)] = jnp.zeros((16,), _F32)
        return c

    lax.fori_loop(0, ROWS_PW // 2, zrow, 0, unroll=False)
    pltpu.sync_copy(stage, acc_sh.at[pl.ds(sid * ROWS_PW, ROWS_PW // 2)])
    pltpu.sync_copy(stage,
                    acc_sh.at[pl.ds(sid * ROWS_PW + ROWS_PW // 2, ROWS_PW // 2)])
    plsc.subcore_barrier()

    def chunk(ch, carry):
        base = ebase + ch * ECH
        pltpu.sync_copy(src_hbm.at[pl.ds(base, ECH)], sidx)
        for j in range(ECH // 16):
            sl = pl.ds(j * 16, 16)
            sidx[sl] = sidx[sl] + roff
        pltpu.async_copy(b_hbm.at[sidx], rows, sem).wait()
        pltpu.sync_copy(dst_hbm.at[pl.ds(base, ECH)], didx)
        pltpu.sync_copy(rows, acc_sh.at[didx], add=True)
        return carry

    lax.fori_loop(0, NFULL, chunk, 0, unroll=False)

    tbase = ebase + NFULL * ECH
    pltpu.sync_copy(src_hbm.at[pl.ds(tbase, ETAIL)], sidx_t)
    for j in range(ETAIL // 16):
        sl = pl.ds(j * 16, 16)
        sidx_t[sl] = sidx_t[sl] + roff
    pltpu.async_copy(b_hbm.at[sidx_t], rows_t, sem).wait()
    pltpu.sync_copy(dst_hbm.at[pl.ds(tbase, ETAIL)], didx_t)
    pltpu.sync_copy(rows_t, acc_sh.at[didx_t], add=True)

    plsc.subcore_barrier()
    for hh in range(2):
        sl = pl.ds(sid * ROWS_PW + hh * (ROWS_PW // 2), ROWS_PW // 2)
        pltpu.sync_copy(acc_sh.at[sl], stage)
        osl = pl.ds(roff + sid * ROWS_PW + hh * (ROWS_PW // 2), ROWS_PW // 2)
        pltpu.sync_copy(stage, out_hbm.at[osl])


def _segsum(src, dst, b_hm):
    mesh = plsc.VectorSubcoreMesh(core_axis_name="c", subcore_axis_name="s",
                                  num_cores=NC, num_subcores=NS)
    f = pl.kernel(
        _seg_body, mesh=mesh,
        compiler_params=pltpu.CompilerParams(use_tc_tiling_on_sc=False),
        out_type=jax.ShapeDtypeStruct((2 * NPAD, DH), _F32),
        scratch_types=[
            pltpu.VMEM((ECH,), jnp.int32),
            pltpu.VMEM((ECH,), jnp.int32),
            pltpu.VMEM((ECH, DH), _F32),
            pltpu.VMEM((ETAIL,), jnp.int32),
            pltpu.VMEM((ETAIL,), jnp.int32),
            pltpu.VMEM((ETAIL, DH), _F32),
            pltpu.VMEM((ROWS_PW // 2, DH), _F32),
            pltpu.VMEM_SHARED((NPAD, DH), _F32),
            pltpu.SemaphoreType.DMA,
        ],
    )
    return f(src, dst, b_hm)


def _l0_body(vf_ref, pos_ref, proj_ref, w0f, w0p, w0c, w1f, w1p, w1c,
             a_ref, b_ref):
    vf = vf_ref[...]
    pos = pos_ref[...]
    proj = proj_ref[...]
    a_ref[...] = _dot(vf, w0f[...]) + _dot(pos, w0p[...]) + _dot(proj, w0c[...])
    b = _dot(vf, w1f[...]) + _dot(pos, w1p[...]) + _dot(proj, w1c[...])
    b_ref[0, :, :] = b[:, 0:DH]
    b_ref[1, :, :] = b[:, DH:D]


def _layer0(vfeat, pos8, proj, w0f, w0p, w0c, w1f, w1p, w1c, bm=2048):
    grid = (NPAD // bm,)
    wspec = lambda shp: pl.BlockSpec(shp, lambda i: (0, 0))
    return pl.pallas_call(
        _l0_body,
        grid=grid,
        in_specs=[
            pl.BlockSpec((bm, D), lambda i: (i, 0)),
            pl.BlockSpec((bm, 8), lambda i: (i, 0)),
            pl.BlockSpec((bm, D), lambda i: (i, 0)),
            wspec((D, D)), wspec((8, D)), wspec((D, D)),
            wspec((D, D)), wspec((8, D)), wspec((D, D)),
        ],
        out_specs=(pl.BlockSpec((bm, D), lambda i: (i, 0)),
                   pl.BlockSpec((2, bm, DH), lambda i: (0, i, 0))),
        out_shape=(jax.ShapeDtypeStruct((NPAD, D), _F32),
                   jax.ShapeDtypeStruct((2, NPAD, DH), _F32)),
    )(vfeat, pos8, proj, w0f, w0p, w0c, w1f, w1p, w1c)


def _lk_body(aprev_ref, plo_ref, phi_ref, pos_ref, w0f, w0p, w1f, w1p,
             a_ref, b_ref):
    ap = aprev_ref[...]
    nfl = jnp.maximum(ap[:, 0:DH] + plo_ref[...], 0.0)
    nfh = jnp.maximum(ap[:, DH:D] + phi_ref[...], 0.0)
    pos = pos_ref[...]
    a_ref[...] = (_dot(nfl, w0f[0:DH, :]) + _dot(nfh, w0f[DH:D, :])
                  + _dot(pos, w0p[...]))
    b = (_dot(nfl, w1f[0:DH, :]) + _dot(nfh, w1f[DH:D, :])
         + _dot(pos, w1p[...]))
    b_ref[0, :, :] = b[:, 0:DH]
    b_ref[1, :, :] = b[:, DH:D]


def _layerk(aprev, parts, pos8, w0f, w0p, w1f, w1p, bm=2048):
    grid = (NPAD // bm,)
    nb = NPAD // bm
    wspec = lambda shp: pl.BlockSpec(shp, lambda i: (0, 0))
    return pl.pallas_call(
        _lk_body,
        grid=grid,
        in_specs=[
            pl.BlockSpec((bm, D), lambda i: (i, 0)),
            pl.BlockSpec((bm, DH), lambda i: (i, 0)),
            pl.BlockSpec((bm, DH), lambda i: (i + nb, 0)),
            pl.BlockSpec((bm, 8), lambda i: (i, 0)),
            wspec((D, D)), wspec((8, D)), wspec((D, D)), wspec((8, D)),
        ],
        out_specs=(pl.BlockSpec((bm, D), lambda i: (i, 0)),
                   pl.BlockSpec((2, bm, DH), lambda i: (0, i, 0))),
        out_shape=(jax.ShapeDtypeStruct((NPAD, D), _F32),
                   jax.ShapeDtypeStruct((2, NPAD, DH), _F32)),
    )(aprev, parts, parts, pos8, w0f, w0p, w1f, w1p)


def _fin_body(aprev_ref, plo_ref, phi_ref, pos_ref, wl1t, nf_ref, np_ref):
    ap = aprev_ref[...]
    nfl = jnp.maximum(ap[:, 0:DH] + plo_ref[...], 0.0)
    nfh = jnp.maximum(ap[:, DH:D] + phi_ref[...], 0.0)
    nf_ref[:, 0:DH] = nfl
    nf_ref[:, DH:D] = nfh
    np_ref[...] = pos_ref[...] + jnp.tanh(
        _dot(nfl, wl1t[0:DH, :]) + _dot(nfh, wl1t[DH:D, :]))


def _final(aprev, parts, pos8, wl1t, bm=2048):
    grid = (NPAD // bm,)
    nb = NPAD // bm
    wspec = lambda shp: pl.BlockSpec(shp, lambda i: (0, 0))
    return pl.pallas_call(
        _fin_body,
        grid=grid,
        in_specs=[
            pl.BlockSpec((bm, D), lambda i: (i, 0)),
            pl.BlockSpec((bm, DH), lambda i: (i, 0)),
            pl.BlockSpec((bm, DH), lambda i: (i + nb, 0)),
            pl.BlockSpec((bm, 8), lambda i: (i, 0)),
            wspec((D, 8)),
        ],
        out_specs=(pl.BlockSpec((bm, D), lambda i: (i, 0)),
                   pl.BlockSpec((bm, 8), lambda i: (i, 0))),
        out_shape=(jax.ShapeDtypeStruct((NPAD, D), _F32),
                   jax.ShapeDtypeStruct((NPAD, 8), _F32)),
    )(aprev, parts, parts, pos8, wl1t)


def kernel(vertex_positions, vertex_features, edge_index, feat0, feat1,
           feat2, feat3, W_lin0, w0_g0, w1_g0, w0_g1, w1_g1, w0_g2, w1_g2,
           W_lin1):
    f32 = _F32
    npadv = NPAD - N
    post = jnp.concatenate(
        [vertex_positions.T, jnp.ones((3, npadv), f32)], axis=1)
    pos8 = jnp.pad(vertex_positions, ((0, npadv), (0, 5)))
    vfeat = jnp.pad(vertex_features, ((0, npadv), (0, 0)))
    fts = []
    for f, c, sz, pp in zip((feat0, feat1, feat2, feat3), CHANS, SIZES, PPAD):
        ft = f.reshape(c, sz * sz).T
        fts.append(jnp.pad(ft, ((0, pp - sz * sz), (0, 0))))
    wt = W_lin0.T

    def wpad3(wm):
        wp = jnp.pad(wm[0:3], ((0, 5), (0, 0)))
        return wm[3:], wp

    def wsplit0(wm):
        wp = jnp.pad(wm[D:D + 3], ((0, 5), (0, 0)))
        return wm[0:D], wp, wm[D + 3:]

    w0f, w0p, w0c = wsplit0(w0_g0)
    w1f, w1p, w1c = wsplit0(w1_g0)
    w0f1, w0p1 = wpad3(w0_g1)
    w1f1, w1p1 = wpad3(w1_g1)
    w0f2, w0p2 = wpad3(w0_g2)
    w1f2, w1p2 = wpad3(w1_g2)
    wl1t = jnp.pad(W_lin1.T, ((0, 0), (0, 5)))
    src = edge_index[0]
    dst = edge_index[1]

    t, idx = _prep(post, *fts, wt)
    proj = _proj(t, idx.reshape(-1))
    a0, b0 = _layer0(vfeat, pos8, proj, w0f, w0p, w0c, w1f, w1p, w1c)
    parts = _segsum(src, dst, b0.reshape(2 * NPAD, DH))
    a1, b1 = _layerk(a0, parts, pos8, w0f1, w0p1, w1f1, w1p1)
    parts = _segsum(src, dst, b1.reshape(2 * NPAD, DH))
    a2, b2 = _layerk(a1, parts, pos8, w0f2, w0p2, w1f2, w1p2)
    parts = _segsum(src, dst, b2.reshape(2 * NPAD, DH))
    nf, npos = _final(a2, parts, pos8, wl1t)
    return npos[:N, :3], nf[:N, :]


# speedup vs baseline: 2.7944x; 1.0366x over previous
import functools

import jax
import jax.numpy as jnp
from jax import lax
from jax.experimental import pallas as pl
from jax.experimental.pallas import tpu as pltpu
from jax.experimental.pallas import tpu_sc as plsc

N = 10000
NPAD = 10240
E = 320000
D = 128
SIZES = (56, 28, 14, 7)
CHANS = (256, 512, 1024, 2048)
COFF = (0, 256, 768, 1792, 3840)
PPAD = (3136, 784, 200, 56)
OFFS = (0, 3136, 3920, 4120)
ZROW = 4176
TROWS = 4184

NC, NS = 2, 16
NW = NC * NS
VPW = NPAD // NW
VCH = 80
ECH = 128
NCH = 160
EPT = NCH * ECH
EPAD = NS * EPT
ROWS_PW = NPAD // NS
DH = D // 2
ACC_P = NS * VPW

_PREC = jax.lax.Precision.HIGHEST
_F32 = jnp.float32


def _dot(a, b):
    return jnp.dot(a, b, preferred_element_type=_F32, precision=_PREC)


def _prep_body(f0_ref, f1_ref, f2_ref, f3_ref, wt_ref, t_ref):
    t_ref[0:3136, :] = _dot(f0_ref[...], wt_ref[COFF[0]:COFF[1], :])
    t_ref[3136:3920, :] = _dot(f1_ref[...], wt_ref[COFF[1]:COFF[2], :])
    t_ref[3920:4120, :] = _dot(f2_ref[...], wt_ref[COFF[2]:COFF[3], :])
    t_ref[4120:4176, :] = _dot(f3_ref[...], wt_ref[COFF[3]:COFF[4], :])
    t_ref[4176:4184, :] = jnp.zeros((8, D), _F32)


def _prep(f0t, f1t, f2t, f3t, wt):
    return pl.pallas_call(
        _prep_body,
        out_shape=jax.ShapeDtypeStruct((TROWS, D), _F32),
    )(f0t, f1t, f2t, f3t, wt)


def _gather_indices(vertex_positions):
    z = vertex_positions[:, 2]
    h = 248.0 * (vertex_positions[:, 1] / z) + 111.5
    w = 248.0 * (vertex_positions[:, 0] / (-z)) + 111.5
    h = jnp.clip(h, 0.0, 223.0)
    w = jnp.clip(w, 0.0, 223.0)
    cols = []
    for s in range(4):
        size = SIZES[s]
        x = w / (224.0 / size)
        y = h / (224.0 / size)
        x1 = jnp.floor(x).astype(jnp.int32)
        y1 = jnp.floor(y).astype(jnp.int32)
        x2 = jnp.minimum(jnp.ceil(x).astype(jnp.int32), size - 1)
        y2 = jnp.minimum(jnp.ceil(y).astype(jnp.int32), size - 1)
        xi = x.astype(jnp.int32)
        yi = y.astype(jnp.int32)
        m = ((x2 - xi) * (y2 - yi)) == 1
        lin = OFFS[s] + x1 * size + y1
        lin = jnp.clip(lin, OFFS[s], OFFS[s] + size * size - 1)
        cols.append(jnp.where(m, lin, ZROW))
    idx = jnp.stack(cols)
    return jnp.pad(idx, ((0, 0), (0, NPAD - N)), constant_values=ZROW)


def _proj_body(t_hbm, idx2_hbm, out_hbm,
               idxp, r0, r1, r2, r3, r4, r5, r6, r7, sem_a, sem_b):
    cid = lax.axis_index("c")
    sid = lax.axis_index("s")
    wid = cid * NS + sid
    base = wid * VPW
    set_a = (r0, r1, r2, r3)
    set_b = (r4, r5, r6, r7)

    for s in range(4):
        pltpu.sync_copy(idx2_hbm.at[pl.ds(s * 128 + wid * 4, 4)],
                        idxp.at[pl.ds(s * 4, 4)])

    def issue(k, st, sem):
        for s in range(4):
            pltpu.async_copy(t_hbm.at[idxp.at[s * 4 + k]], st[s], sem)

    def drain(st, sem):
        for s in range(4):
            pltpu.make_async_copy(t_hbm.at[pl.ds(0, VCH)], st[s], sem).wait()

    def write_out(k, st):
        for s in range(4):
            pltpu.sync_copy(st[s],
                            out_hbm.at[s, pl.ds(base + k * VCH, VCH)])

    issue(0, set_a, sem_a)
    issue(1, set_b, sem_b)
    drain(set_a, sem_a)
    write_out(0, set_a)
    issue(2, set_a, sem_a)
    drain(set_b, sem_b)
    write_out(1, set_b)
    issue(3, set_b, sem_b)
    drain(set_a, sem_a)
    write_out(2, set_a)
    drain(set_b, sem_b)
    write_out(3, set_b)


def _proj(t, idx2):
    mesh = plsc.VectorSubcoreMesh(core_axis_name="c", subcore_axis_name="s",
                                  num_cores=NC, num_subcores=NS)
    f = pl.kernel(
        _proj_body, mesh=mesh,
        out_type=jax.ShapeDtypeStruct((4, NPAD, D), _F32),
        scratch_types=[pltpu.VMEM((16, VCH), jnp.int32)]
        + [pltpu.VMEM((VCH, D), _F32)] * 8
        + [pltpu.SemaphoreType.DMA, pltpu.SemaphoreType.DMA],
    )
    return f(t, idx2)


def _seg_body(src2_hbm, dst2_hbm, blo_hbm, bhi_hbm, z_hbm, out_hbm,
              sidx, didx, q0, q1, q2, q3, stage, acc_sh, sem_a, sem_b):
    cid = lax.axis_index("c")
    sid = lax.axis_index("s")
    roff = cid * NPAD
    rows = q0

    pltpu.sync_copy(z_hbm, stage)
    pltpu.sync_copy(stage, acc_sh.at[pl.ds(sid * ROWS_PW, ROWS_PW // 2)])
    pltpu.sync_copy(stage,
                    acc_sh.at[pl.ds(sid * ROWS_PW + ROWS_PW // 2, ROWS_PW // 2)])
    plsc.subcore_barrier()

    def run(b_ref):
        def issue1(ch, buf, sem):
            pltpu.async_copy(b_ref.at[sidx.at[ch]], buf, sem)

        def drain1(buf, sem):
            pltpu.make_async_copy(b_ref.at[pl.ds(0, ECH)], buf, sem).wait()

        issue1(0, q0, sem_a)
        issue1(1, q1, sem_b)

        def pair(h, carry):
            drain1(q0, sem_a)
            pltpu.sync_copy(q0, acc_sh.at[didx.at[2 * h]], add=True)
            issue1(2 * h + 2, q0, sem_a)
            drain1(q1, sem_b)
            pltpu.sync_copy(q1, acc_sh.at[didx.at[2 * h + 1]], add=True)
            issue1(2 * h + 3, q1, sem_b)
            return carry

        lax.fori_loop(0, NCH // 2 - 1, pair, 0)
        drain1(q0, sem_a)
        pltpu.sync_copy(q0, acc_sh.at[didx.at[NCH - 2]], add=True)
        drain1(q1, sem_b)
        pltpu.sync_copy(q1, acc_sh.at[didx.at[NCH - 1]], add=True)

    pltpu.sync_copy(src2_hbm.at[pl.ds(sid * NCH, NCH)], sidx)
    pltpu.sync_copy(dst2_hbm.at[pl.ds(sid * NCH, NCH)], didx)
    pl.when(cid == 0)(lambda: run(blo_hbm))
    pl.when(cid == 1)(lambda: run(bhi_hbm))

    plsc.subcore_barrier()
    for hh in range(2):
        sl = pl.ds(sid * ROWS_PW + hh * (ROWS_PW // 2), ROWS_PW // 2)
        pltpu.sync_copy(acc_sh.at[sl], stage)
        osl = pl.ds(roff + sid * ROWS_PW + hh * (ROWS_PW // 2), ROWS_PW // 2)
        pltpu.sync_copy(stage, out_hbm.at[osl])


def _segsum(src2, dst2, b_lo, b_hi, zseg):
    mesh = plsc.VectorSubcoreMesh(core_axis_name="c", subcore_axis_name="s",
                                  num_cores=NC, num_subcores=NS)
    f = pl.kernel(
        _seg_body, mesh=mesh,
        compiler_params=pltpu.CompilerParams(use_tc_tiling_on_sc=False),
        out_type=jax.ShapeDtypeStruct((2 * NPAD, DH), _F32),
        scratch_types=[
            pltpu.VMEM((NCH, ECH), jnp.int32),
            pltpu.VMEM((NCH, ECH), jnp.int32),
            pltpu.VMEM((ECH, DH), _F32),
            pltpu.VMEM((ECH, DH), _F32),
            pltpu.VMEM((ECH, DH), _F32),
            pltpu.VMEM((ECH, DH), _F32),
            pltpu.VMEM((ROWS_PW // 2, DH), _F32),
            pltpu.VMEM_SHARED((NPAD, DH), _F32),
            pltpu.SemaphoreType.DMA, pltpu.SemaphoreType.DMA,
        ],
    )
    return f(src2, dst2, b_lo, b_hi, zseg)


def _l0_body(vf_ref, pos_ref, proj_ref, w0f, w0p, w0c, w1f, w1p, w1c,
             a_ref, b_ref):
    vf = vf_ref[...]
    pos = pos_ref[...]
    proj = (proj_ref[0, :, :] + proj_ref[1, :, :]
            + proj_ref[2, :, :] + proj_ref[3, :, :])
    a_ref[...] = _dot(vf, w0f[...]) + _dot(pos, w0p[...]) + _dot(proj, w0c[...])
    b = _dot(vf, w1f[...]) + _dot(pos, w1p[...]) + _dot(proj, w1c[...])
    b_ref[0, :, :] = b[:, 0:DH]
    b_ref[1, :, :] = b[:, DH:D]


def _layer0(vfeat, pos8, proj, w0f, w0p, w0c, w1f, w1p, w1c, bm=2048):
    grid = (NPAD // bm,)
    wspec = lambda shp: pl.BlockSpec(shp, lambda i: (0, 0))
    return pl.pallas_call(
        _l0_body,
        grid=grid,
        in_specs=[
            pl.BlockSpec((bm, D), lambda i: (i, 0)),
            pl.BlockSpec((bm, 8), lambda i: (i, 0)),
            pl.BlockSpec((4, bm, D), lambda i: (0, i, 0)),
            wspec((D, D)), wspec((8, D)), wspec((D, D)),
            wspec((D, D)), wspec((8, D)), wspec((D, D)),
        ],
        out_specs=(pl.BlockSpec((bm, D), lambda i: (i, 0)),
                   pl.BlockSpec((2, bm, DH), lambda i: (0, i, 0))),
        out_shape=(jax.ShapeDtypeStruct((NPAD, D), _F32),
                   jax.ShapeDtypeStruct((2, NPAD, DH), _F32)),
    )(vfeat, pos8, proj, w0f, w0p, w0c, w1f, w1p, w1c)


def _lk_body(aprev_ref, plo_ref, phi_ref, pos_ref, w0f, w0p, w1f, w1p,
             a_ref, b_ref):
    ap = aprev_ref[...]
    nfl = jnp.maximum(ap[:, 0:DH] + plo_ref[...], 0.0)
    nfh = jnp.maximum(ap[:, DH:D] + phi_ref[...], 0.0)
    pos = pos_ref[...]
    a_ref[...] = (_dot(nfl, w0f[0:DH, :]) + _dot(nfh, w0f[DH:D, :])
                  + _dot(pos, w0p[...]))
    b = (_dot(nfl, w1f[0:DH, :]) + _dot(nfh, w1f[DH:D, :])
         + _dot(pos, w1p[...]))
    b_ref[0, :, :] = b[:, 0:DH]
    b_ref[1, :, :] = b[:, DH:D]


def _layerk(aprev, parts, pos8, w0f, w0p, w1f, w1p, bm=2048):
    grid = (NPAD // bm,)
    nb = NPAD // bm
    wspec = lambda shp: pl.BlockSpec(shp, lambda i: (0, 0))
    return pl.pallas_call(
        _lk_body,
        grid=grid,
        in_specs=[
            pl.BlockSpec((bm, D), lambda i: (i, 0)),
            pl.BlockSpec((bm, DH), lambda i: (i, 0)),
            pl.BlockSpec((bm, DH), lambda i: (i + nb, 0)),
            pl.BlockSpec((bm, 8), lambda i: (i, 0)),
            wspec((D, D)), wspec((8, D)), wspec((D, D)), wspec((8, D)),
        ],
        out_specs=(pl.BlockSpec((bm, D), lambda i: (i, 0)),
                   pl.BlockSpec((2, bm, DH), lambda i: (0, i, 0))),
        out_shape=(jax.ShapeDtypeStruct((NPAD, D), _F32),
                   jax.ShapeDtypeStruct((2, NPAD, DH), _F32)),
    )(aprev, parts, parts, pos8, w0f, w0p, w1f, w1p)


def _fin_body(aprev_ref, plo_ref, phi_ref, pos_ref, wl1t, nf_ref, np_ref):
    ap = aprev_ref[...]
    nfl = jnp.maximum(ap[:, 0:DH] + plo_ref[...], 0.0)
    nfh = jnp.maximum(ap[:, DH:D] + phi_ref[...], 0.0)
    nf_ref[:, 0:DH] = nfl
    nf_ref[:, DH:D] = nfh
    np_ref[...] = pos_ref[...] + jnp.tanh(
        _dot(nfl, wl1t[0:DH, :]) + _dot(nfh, wl1t[DH:D, :]))


def _final(aprev, parts, pos8, wl1t, bm=2048):
    grid = (NPAD // bm,)
    nb = NPAD // bm
    wspec = lambda shp: pl.BlockSpec(shp, lambda i: (0, 0))
    return pl.pallas_call(
        _fin_body,
        grid=grid,
        in_specs=[
            pl.BlockSpec((bm, D), lambda i: (i, 0)),
            pl.BlockSpec((bm, DH), lambda i: (i, 0)),
            pl.BlockSpec((bm, DH), lambda i: (i + nb, 0)),
            pl.BlockSpec((bm, 8), lambda i: (i, 0)),
            wspec((D, 8)),
        ],
        out_specs=(pl.BlockSpec((bm, D), lambda i: (i, 0)),
                   pl.BlockSpec((bm, 8), lambda i: (i, 0))),
        out_shape=(jax.ShapeDtypeStruct((NPAD, D), _F32),
                   jax.ShapeDtypeStruct((NPAD, 8), _F32)),
    )(aprev, parts, parts, pos8, wl1t)


def kernel(vertex_positions, vertex_features, edge_index, feat0, feat1,
           feat2, feat3, W_lin0, w0_g0, w1_g0, w0_g1, w1_g1, w0_g2, w1_g2,
           W_lin1):
    f32 = _F32
    npadv = NPAD - N
    pos8 = jnp.pad(vertex_positions, ((0, npadv), (0, 5)))
    vfeat = jnp.pad(vertex_features, ((0, npadv), (0, 0)))
    fts = []
    for f, c, sz, pp in zip((feat0, feat1, feat2, feat3), CHANS, SIZES, PPAD):
        ft = f.reshape(c, sz * sz).T
        fts.append(jnp.pad(ft, ((0, pp - sz * sz), (0, 0))))
    wt = W_lin0.T

    def wpad3(wm):
        wp = jnp.pad(wm[0:3], ((0, 5), (0, 0)))
        return wm[3:], wp

    def wsplit0(wm):
        wp = jnp.pad(wm[D:D + 3], ((0, 5), (0, 0)))
        return wm[0:D], wp, wm[D + 3:]

    w0f, w0p, w0c = wsplit0(w0_g0)
    w1f, w1p, w1c = wsplit0(w1_g0)
    w0f1, w0p1 = wpad3(w0_g1)
    w1f1, w1p1 = wpad3(w1_g1)
    w0f2, w0p2 = wpad3(w0_g2)
    w1f2, w1p2 = wpad3(w1_g2)
    wl1t = jnp.pad(W_lin1.T, ((0, 0), (0, 5)))
    npade = EPAD - E
    src2 = jnp.concatenate(
        [edge_index[0], jnp.zeros((npade,), jnp.int32)]).reshape(-1, ECH)
    dst2 = jnp.concatenate(
        [edge_index[1], jnp.full((npade,), NPAD - 1, jnp.int32)]
    ).reshape(-1, ECH)
    zseg = jnp.zeros((ROWS_PW // 2, DH), f32)

    idx = _gather_indices(vertex_positions)

    t = _prep(*fts, wt)
    proj = _proj(t, idx.reshape(-1, VCH))
    a0, b0 = _layer0(vfeat, pos8, proj, w0f, w0p, w0c, w1f, w1p, w1c)
    parts = _segsum(src2, dst2, b0[0], b0[1], zseg)
    a1, b1 = _layerk(a0, parts, pos8, w0f1, w0p1, w1f1, w1p1)
    parts = _segsum(src2, dst2, b1[0], b1[1], zseg)
    a2, b2 = _layerk(a1, parts, pos8, w0f2, w0p2, w1f2, w1p2)
    parts = _segsum(src2, dst2, b2[0], b2[1], zseg)
    nf, npos = _final(a2, parts, pos8, wl1t)
    return npos[:N, :3], nf[:N, :]


# speedup vs baseline: 2.8128x; 1.0066x over previous
import functools

import jax
import jax.numpy as jnp
from jax import lax
from jax.experimental import pallas as pl
from jax.experimental.pallas import tpu as pltpu
from jax.experimental.pallas import tpu_sc as plsc

N = 10000
NPAD = 10240
E = 320000
D = 128
SIZES = (56, 28, 14, 7)
CHANS = (256, 512, 1024, 2048)
COFF = (0, 256, 768, 1792, 3840)
PPAD = (3136, 784, 200, 56)
OFFS = (0, 3136, 3920, 4120)
ZROW = 4176
TROWS = 4184

NC, NS = 2, 16
NW = NC * NS
VPW = NPAD // NW
VCH = 80
ECH = 128
NCH = 160
EPT = NCH * ECH
EPAD = NS * EPT
ROWS_PW = NPAD // NS
DH = D // 2
ACC_P = NS * VPW

_PREC = None
_F32 = jnp.float32


def _dot(a, b):
    return jnp.dot(a, b, preferred_element_type=_F32, precision=_PREC)


def _prep_body(f0_ref, f1_ref, f2_ref, f3_ref, wt_ref, t_ref):
    t_ref[0:3136, :] = _dot(f0_ref[...], wt_ref[COFF[0]:COFF[1], :])
    t_ref[3136:3920, :] = _dot(f1_ref[...], wt_ref[COFF[1]:COFF[2], :])
    t_ref[3920:4120, :] = _dot(f2_ref[...], wt_ref[COFF[2]:COFF[3], :])
    t_ref[4120:4176, :] = _dot(f3_ref[...], wt_ref[COFF[3]:COFF[4], :])
    t_ref[4176:4184, :] = jnp.zeros((8, D), _F32)


def _prep(f0t, f1t, f2t, f3t, wt):
    return pl.pallas_call(
        _prep_body,
        out_shape=jax.ShapeDtypeStruct((TROWS, D), _F32),
    )(f0t, f1t, f2t, f3t, wt)


def _gather_indices(vertex_positions):
    z = vertex_positions[:, 2]
    h = 248.0 * (vertex_positions[:, 1] / z) + 111.5
    w = 248.0 * (vertex_positions[:, 0] / (-z)) + 111.5
    h = jnp.clip(h, 0.0, 223.0)
    w = jnp.clip(w, 0.0, 223.0)
    cols = []
    for s in range(4):
        size = SIZES[s]
        x = w / (224.0 / size)
        y = h / (224.0 / size)
        x1 = jnp.floor(x).astype(jnp.int32)
        y1 = jnp.floor(y).astype(jnp.int32)
        x2 = jnp.minimum(jnp.ceil(x).astype(jnp.int32), size - 1)
        y2 = jnp.minimum(jnp.ceil(y).astype(jnp.int32), size - 1)
        xi = x.astype(jnp.int32)
        yi = y.astype(jnp.int32)
        m = ((x2 - xi) * (y2 - yi)) == 1
        lin = OFFS[s] + x1 * size + y1
        lin = jnp.clip(lin, OFFS[s], OFFS[s] + size * size - 1)
        cols.append(jnp.where(m, lin, ZROW))
    idx = jnp.stack(cols)
    return jnp.pad(idx, ((0, 0), (0, NPAD - N)), constant_values=ZROW)


def _proj_body(t_hbm, idx2_hbm, out_hbm,
               idxp, r0, r1, r2, r3, r4, r5, r6, r7, sem_a, sem_b):
    cid = lax.axis_index("c")
    sid = lax.axis_index("s")
    wid = cid * NS + sid
    base = wid * VPW
    set_a = (r0, r1, r2, r3)
    set_b = (r4, r5, r6, r7)

    for s in range(4):
        pltpu.sync_copy(idx2_hbm.at[pl.ds(s * 128 + wid * 4, 4)],
                        idxp.at[pl.ds(s * 4, 4)])

    def issue(k, st, sem):
        for s in range(4):
            pltpu.async_copy(t_hbm.at[idxp.at[s * 4 + k]], st[s], sem)

    def drain(st, sem):
        for s in range(4):
            pltpu.make_async_copy(t_hbm.at[pl.ds(0, VCH)], st[s], sem).wait()

    def write_out(k, st):
        for s in range(4):
            pltpu.sync_copy(st[s],
                            out_hbm.at[s, pl.ds(base + k * VCH, VCH)])

    issue(0, set_a, sem_a)
    issue(1, set_b, sem_b)
    drain(set_a, sem_a)
    write_out(0, set_a)
    issue(2, set_a, sem_a)
    drain(set_b, sem_b)
    write_out(1, set_b)
    issue(3, set_b, sem_b)
    drain(set_a, sem_a)
    write_out(2, set_a)
    drain(set_b, sem_b)
    write_out(3, set_b)


def _proj(t, idx2):
    mesh = plsc.VectorSubcoreMesh(core_axis_name="c", subcore_axis_name="s",
                                  num_cores=NC, num_subcores=NS)
    f = pl.kernel(
        _proj_body, mesh=mesh,
        out_type=jax.ShapeDtypeStruct((4, NPAD, D), _F32),
        scratch_types=[pltpu.VMEM((16, VCH), jnp.int32)]
        + [pltpu.VMEM((VCH, D), _F32)] * 8
        + [pltpu.SemaphoreType.DMA, pltpu.SemaphoreType.DMA],
    )
    return f(t, idx2)


def _seg_body(src2_hbm, dst2_hbm, blo_hbm, bhi_hbm, z_hbm, out_hbm,
              sidx, didx, q0, q1, q2, q3, stage, acc_sh, sem_a, sem_b):
    cid = lax.axis_index("c")
    sid = lax.axis_index("s")
    roff = cid * NPAD
    rows = q0

    pltpu.sync_copy(z_hbm, stage)
    pltpu.sync_copy(stage, acc_sh.at[pl.ds(sid * ROWS_PW, ROWS_PW // 2)])
    pltpu.sync_copy(stage,
                    acc_sh.at[pl.ds(sid * ROWS_PW + ROWS_PW // 2, ROWS_PW // 2)])
    plsc.subcore_barrier()

    def run(b_ref):
        def issue1(ch, buf, sem):
            pltpu.async_copy(b_ref.at[sidx.at[ch]], buf, sem)

        def drain1(buf, sem):
            pltpu.make_async_copy(b_ref.at[pl.ds(0, ECH)], buf, sem).wait()

        issue1(0, q0, sem_a)
        issue1(1, q1, sem_b)

        def pair(h, carry):
            drain1(q0, sem_a)
            pltpu.sync_copy(q0, acc_sh.at[didx.at[2 * h]], add=True)
            issue1(2 * h + 2, q0, sem_a)
            drain1(q1, sem_b)
            pltpu.sync_copy(q1, acc_sh.at[didx.at[2 * h + 1]], add=True)
            issue1(2 * h + 3, q1, sem_b)
            return carry

        lax.fori_loop(0, NCH // 2 - 1, pair, 0)
        drain1(q0, sem_a)
        pltpu.sync_copy(q0, acc_sh.at[didx.at[NCH - 2]], add=True)
        drain1(q1, sem_b)
        pltpu.sync_copy(q1, acc_sh.at[didx.at[NCH - 1]], add=True)

    pltpu.sync_copy(src2_hbm.at[pl.ds(sid * NCH, NCH)], sidx)
    pltpu.sync_copy(dst2_hbm.at[pl.ds(sid * NCH, NCH)], didx)
    pl.when(cid == 0)(lambda: run(blo_hbm))
    pl.when(cid == 1)(lambda: run(bhi_hbm))

    plsc.subcore_barrier()
    for hh in range(2):
        sl = pl.ds(sid * ROWS_PW + hh * (ROWS_PW // 2), ROWS_PW // 2)
        pltpu.sync_copy(acc_sh.at[sl], stage)
        osl = pl.ds(roff + sid * ROWS_PW + hh * (ROWS_PW // 2), ROWS_PW // 2)
        pltpu.sync_copy(stage, out_hbm.at[osl])


def _segsum(src2, dst2, b_lo, b_hi, zseg):
    mesh = plsc.VectorSubcoreMesh(core_axis_name="c", subcore_axis_name="s",
                                  num_cores=NC, num_subcores=NS)
    f = pl.kernel(
        _seg_body, mesh=mesh,
        compiler_params=pltpu.CompilerParams(use_tc_tiling_on_sc=False),
        out_type=jax.ShapeDtypeStruct((2 * NPAD, DH), _F32),
        scratch_types=[
            pltpu.VMEM((NCH, ECH), jnp.int32),
            pltpu.VMEM((NCH, ECH), jnp.int32),
            pltpu.VMEM((ECH, DH), _F32),
            pltpu.VMEM((ECH, DH), _F32),
            pltpu.VMEM((ECH, DH), _F32),
            pltpu.VMEM((ECH, DH), _F32),
            pltpu.VMEM((ROWS_PW // 2, DH), _F32),
            pltpu.VMEM_SHARED((NPAD, DH), _F32),
            pltpu.SemaphoreType.DMA, pltpu.SemaphoreType.DMA,
        ],
    )
    return f(src2, dst2, b_lo, b_hi, zseg)


def _l0_body(vf_ref, pos_ref, proj_ref, w0f, w0p, w0c, w1f, w1p, w1c,
             a_ref, b_ref):
    vf = vf_ref[...]
    pos = pos_ref[...]
    proj = (proj_ref[0, :, :] + proj_ref[1, :, :]
            + proj_ref[2, :, :] + proj_ref[3, :, :])
    a_ref[...] = _dot(vf, w0f[...]) + _dot(pos, w0p[...]) + _dot(proj, w0c[...])
    b = _dot(vf, w1f[...]) + _dot(pos, w1p[...]) + _dot(proj, w1c[...])
    b_ref[0, :, :] = b[:, 0:DH]
    b_ref[1, :, :] = b[:, DH:D]


def _layer0(vfeat, pos8, proj, w0f, w0p, w0c, w1f, w1p, w1c, bm=2048):
    grid = (NPAD // bm,)
    wspec = lambda shp: pl.BlockSpec(shp, lambda i: (0, 0))
    return pl.pallas_call(
        _l0_body,
        grid=grid,
        in_specs=[
            pl.BlockSpec((bm, D), lambda i: (i, 0)),
            pl.BlockSpec((bm, 8), lambda i: (i, 0)),
            pl.BlockSpec((4, bm, D), lambda i: (0, i, 0)),
            wspec((D, D)), wspec((8, D)), wspec((D, D)),
            wspec((D, D)), wspec((8, D)), wspec((D, D)),
        ],
        out_specs=(pl.BlockSpec((bm, D), lambda i: (i, 0)),
                   pl.BlockSpec((2, bm, DH), lambda i: (0, i, 0))),
        out_shape=(jax.ShapeDtypeStruct((NPAD, D), _F32),
                   jax.ShapeDtypeStruct((2, NPAD, DH), _F32)),
    )(vfeat, pos8, proj, w0f, w0p, w0c, w1f, w1p, w1c)


def _lk_body(aprev_ref, plo_ref, phi_ref, pos_ref, w0f, w0p, w1f, w1p,
             a_ref, b_ref):
    ap = aprev_ref[...]
    nfl = jnp.maximum(ap[:, 0:DH] + plo_ref[...], 0.0)
    nfh = jnp.maximum(ap[:, DH:D] + phi_ref[...], 0.0)
    pos = pos_ref[...]
    a_ref[...] = (_dot(nfl, w0f[0:DH, :]) + _dot(nfh, w0f[DH:D, :])
                  + _dot(pos, w0p[...]))
    b = (_dot(nfl, w1f[0:DH, :]) + _dot(nfh, w1f[DH:D, :])
         + _dot(pos, w1p[...]))
    b_ref[0, :, :] = b[:, 0:DH]
    b_ref[1, :, :] = b[:, DH:D]


def _layerk(aprev, parts, pos8, w0f, w0p, w1f, w1p, bm=2048):
    grid = (NPAD // bm,)
    nb = NPAD // bm
    wspec = lambda shp: pl.BlockSpec(shp, lambda i: (0, 0))
    return pl.pallas_call(
        _lk_body,
        grid=grid,
        in_specs=[
            pl.BlockSpec((bm, D), lambda i: (i, 0)),
            pl.BlockSpec((bm, DH), lambda i: (i, 0)),
            pl.BlockSpec((bm, DH), lambda i: (i + nb, 0)),
            pl.BlockSpec((bm, 8), lambda i: (i, 0)),
            wspec((D, D)), wspec((8, D)), wspec((D, D)), wspec((8, D)),
        ],
        out_specs=(pl.BlockSpec((bm, D), lambda i: (i, 0)),
                   pl.BlockSpec((2, bm, DH), lambda i: (0, i, 0))),
        out_shape=(jax.ShapeDtypeStruct((NPAD, D), _F32),
                   jax.ShapeDtypeStruct((2, NPAD, DH), _F32)),
    )(aprev, parts, parts, pos8, w0f, w0p, w1f, w1p)


def _fin_body(aprev_ref, plo_ref, phi_ref, pos_ref, wl1t, nf_ref, np_ref):
    ap = aprev_ref[...]
    nfl = jnp.maximum(ap[:, 0:DH] + plo_ref[...], 0.0)
    nfh = jnp.maximum(ap[:, DH:D] + phi_ref[...], 0.0)
    nf_ref[:, 0:DH] = nfl
    nf_ref[:, DH:D] = nfh
    np_ref[...] = pos_ref[...] + jnp.tanh(
        _dot(nfl, wl1t[0:DH, :]) + _dot(nfh, wl1t[DH:D, :]))


def _final(aprev, parts, pos8, wl1t, bm=2048):
    grid = (NPAD // bm,)
    nb = NPAD // bm
    wspec = lambda shp: pl.BlockSpec(shp, lambda i: (0, 0))
    return pl.pallas_call(
        _fin_body,
        grid=grid,
        in_specs=[
            pl.BlockSpec((bm, D), lambda i: (i, 0)),
            pl.BlockSpec((bm, DH), lambda i: (i, 0)),
            pl.BlockSpec((bm, DH), lambda i: (i + nb, 0)),
            pl.BlockSpec((bm, 8), lambda i: (i, 0)),
            wspec((D, 8)),
        ],
        out_specs=(pl.BlockSpec((bm, D), lambda i: (i, 0)),
                   pl.BlockSpec((bm, 8), lambda i: (i, 0))),
        out_shape=(jax.ShapeDtypeStruct((NPAD, D), _F32),
                   jax.ShapeDtypeStruct((NPAD, 8), _F32)),
    )(aprev, parts, parts, pos8, wl1t)


def kernel(vertex_positions, vertex_features, edge_index, feat0, feat1,
           feat2, feat3, W_lin0, w0_g0, w1_g0, w0_g1, w1_g1, w0_g2, w1_g2,
           W_lin1):
    f32 = _F32
    npadv = NPAD - N
    pos8 = jnp.pad(vertex_positions, ((0, npadv), (0, 5)))
    vfeat = jnp.pad(vertex_features, ((0, npadv), (0, 0)))
    fts = []
    for f, c, sz, pp in zip((feat0, feat1, feat2, feat3), CHANS, SIZES, PPAD):
        ft = f.reshape(c, sz * sz).T
        fts.append(jnp.pad(ft, ((0, pp - sz * sz), (0, 0))))
    wt = W_lin0.T

    def wpad3(wm):
        wp = jnp.pad(wm[0:3], ((0, 5), (0, 0)))
        return wm[3:], wp

    def wsplit0(wm):
        wp = jnp.pad(wm[D:D + 3], ((0, 5), (0, 0)))
        return wm[0:D], wp, wm[D + 3:]

    w0f, w0p, w0c = wsplit0(w0_g0)
    w1f, w1p, w1c = wsplit0(w1_g0)
    w0f1, w0p1 = wpad3(w0_g1)
    w1f1, w1p1 = wpad3(w1_g1)
    w0f2, w0p2 = wpad3(w0_g2)
    w1f2, w1p2 = wpad3(w1_g2)
    wl1t = jnp.pad(W_lin1.T, ((0, 0), (0, 5)))
    npade = EPAD - E
    src2 = jnp.concatenate(
        [edge_index[0], jnp.zeros((npade,), jnp.int32)]).reshape(-1, ECH)
    dst2 = jnp.concatenate(
        [edge_index[1], jnp.full((npade,), NPAD - 1, jnp.int32)]
    ).reshape(-1, ECH)
    zseg = jnp.zeros((ROWS_PW // 2, DH), f32)

    idx = _gather_indices(vertex_positions)

    t = _prep(*fts, wt)
    proj = _proj(t, idx.reshape(-1, VCH))
    a0, b0 = _layer0(vfeat, pos8, proj, w0f, w0p, w0c, w1f, w1p, w1c)
    parts = _segsum(src2, dst2, b0[0], b0[1], zseg)
    a1, b1 = _layerk(a0, parts, pos8, w0f1, w0p1, w1f1, w1p1)
    parts = _segsum(src2, dst2, b1[0], b1[1], zseg)
    a2, b2 = _layerk(a1, parts, pos8, w0f2, w0p2, w1f2, w1p2)
    parts = _segsum(src2, dst2, b2[0], b2[1], zseg)
    nf, npos = _final(a2, parts, pos8, wl1t)
    return npos[:N, :3], nf[:N, :]


# speedup vs baseline: 2.9329x; 1.0427x over previous
import functools

import jax
import jax.numpy as jnp
from jax import lax
from jax.experimental import pallas as pl
from jax.experimental.pallas import tpu as pltpu
from jax.experimental.pallas import tpu_sc as plsc

N = 10000
NPAD = 10240
E = 320000
D = 128
SIZES = (56, 28, 14, 7)
CHANS = (256, 512, 1024, 2048)
COFF = (0, 256, 768, 1792, 3840)
PPAD = (3136, 784, 200, 56)
OFFS = (0, 3136, 3920, 4120)
ZROW = 4176
TROWS = 4184

NC, NS = 2, 16
NW = NC * NS
VPW = NPAD // NW
VCH = 80
ECH = 128
NCH = 160
EPT = NCH * ECH
EPAD = NS * EPT
ROWS_PW = NPAD // NS
DH = D // 2
ACC_P = NS * VPW

_PREC = None
_F32 = jnp.float32


def _dot(a, b):
    return jnp.dot(a, b, preferred_element_type=_F32, precision=_PREC)


def _prep_body(f0_ref, f1_ref, f2_ref, f3_ref, wt_ref, t_ref):
    t_ref[0:3136, :] = _dot(f0_ref[...], wt_ref[COFF[0]:COFF[1], :])
    t_ref[3136:3920, :] = _dot(f1_ref[...], wt_ref[COFF[1]:COFF[2], :])
    t_ref[3920:4120, :] = _dot(f2_ref[...], wt_ref[COFF[2]:COFF[3], :])
    t_ref[4120:4176, :] = _dot(f3_ref[...], wt_ref[COFF[3]:COFF[4], :])
    t_ref[4176:4184, :] = jnp.zeros((8, D), _F32)


def _prep(f0t, f1t, f2t, f3t, wt):
    return pl.pallas_call(
        _prep_body,
        out_shape=jax.ShapeDtypeStruct((TROWS, D), _F32),
    )(f0t, f1t, f2t, f3t, wt)


def _gather_indices(vertex_positions):
    z = vertex_positions[:, 2]
    h = 248.0 * (vertex_positions[:, 1] / z) + 111.5
    w = 248.0 * (vertex_positions[:, 0] / (-z)) + 111.5
    h = jnp.clip(h, 0.0, 223.0)
    w = jnp.clip(w, 0.0, 223.0)
    cols = []
    for s in range(4):
        size = SIZES[s]
        x = w / (224.0 / size)
        y = h / (224.0 / size)
        x1 = jnp.floor(x).astype(jnp.int32)
        y1 = jnp.floor(y).astype(jnp.int32)
        x2 = jnp.minimum(jnp.ceil(x).astype(jnp.int32), size - 1)
        y2 = jnp.minimum(jnp.ceil(y).astype(jnp.int32), size - 1)
        xi = x.astype(jnp.int32)
        yi = y.astype(jnp.int32)
        m = ((x2 - xi) * (y2 - yi)) == 1
        lin = OFFS[s] + x1 * size + y1
        lin = jnp.clip(lin, OFFS[s], OFFS[s] + size * size - 1)
        cols.append(jnp.where(m, lin, ZROW))
    idx = jnp.stack(cols)
    return jnp.pad(idx, ((0, 0), (0, NPAD - N)), constant_values=ZROW)


def _proj_body(t_hbm, idx2_hbm, out_hbm,
               idxp, r0, r1, r2, r3, r4, r5, r6, r7, sem_a, sem_b):
    cid = lax.axis_index("c")
    sid = lax.axis_index("s")
    wid = cid * NS + sid
    base = wid * VPW
    set_a = (r0, r1, r2, r3)
    set_b = (r4, r5, r6, r7)

    for s in range(4):
        pltpu.sync_copy(idx2_hbm.at[pl.ds(s * 128 + wid * 4, 4)],
                        idxp.at[pl.ds(s * 4, 4)])

    def issue(k, st, sem):
        for s in range(4):
            pltpu.async_copy(t_hbm.at[idxp.at[s * 4 + k]], st[s], sem)

    def drain(st, sem):
        for s in range(4):
            pltpu.make_async_copy(t_hbm.at[pl.ds(0, VCH)], st[s], sem).wait()

    def write_out(k, st):
        for s in range(4):
            pltpu.sync_copy(st[s],
                            out_hbm.at[s, pl.ds(base + k * VCH, VCH)])

    issue(0, set_a, sem_a)
    issue(1, set_b, sem_b)
    drain(set_a, sem_a)
    write_out(0, set_a)
    issue(2, set_a, sem_a)
    drain(set_b, sem_b)
    write_out(1, set_b)
    issue(3, set_b, sem_b)
    drain(set_a, sem_a)
    write_out(2, set_a)
    drain(set_b, sem_b)
    write_out(3, set_b)


def _proj(t, idx2):
    mesh = plsc.VectorSubcoreMesh(core_axis_name="c", subcore_axis_name="s",
                                  num_cores=NC, num_subcores=NS)
    f = pl.kernel(
        _proj_body, mesh=mesh,
        compiler_params=pltpu.CompilerParams(use_tc_tiling_on_sc=False),
        out_type=jax.ShapeDtypeStruct((4, NPAD, D), _F32),
        scratch_types=[pltpu.VMEM((16, VCH), jnp.int32)]
        + [pltpu.VMEM((VCH, D), _F32)] * 8
        + [pltpu.SemaphoreType.DMA, pltpu.SemaphoreType.DMA],
    )
    return f(t, idx2)


def _seg_body(src2_hbm, dst2_hbm, blo_hbm, bhi_hbm, z_hbm, out_hbm,
              sidx, didx, q0, q1, q2, q3, stage, acc_sh, sem_a, sem_b):
    cid = lax.axis_index("c")
    sid = lax.axis_index("s")
    roff = cid * NPAD
    rows = q0

    pltpu.sync_copy(z_hbm, stage)
    pltpu.sync_copy(stage, acc_sh.at[pl.ds(sid * ROWS_PW, ROWS_PW // 2)])
    pltpu.sync_copy(stage,
                    acc_sh.at[pl.ds(sid * ROWS_PW + ROWS_PW // 2, ROWS_PW // 2)])
    plsc.subcore_barrier()

    def run(b_ref):
        def issue1(ch, buf, sem):
            pltpu.async_copy(b_ref.at[sidx.at[ch]], buf, sem)

        def drain1(buf, sem):
            pltpu.make_async_copy(b_ref.at[pl.ds(0, ECH)], buf, sem).wait()

        issue1(0, q0, sem_a)
        issue1(1, q1, sem_b)

        def pair(h, carry):
            drain1(q0, sem_a)
            pltpu.sync_copy(q0, acc_sh.at[didx.at[2 * h]], add=True)
            issue1(2 * h + 2, q0, sem_a)
            drain1(q1, sem_b)
            pltpu.sync_copy(q1, acc_sh.at[didx.at[2 * h + 1]], add=True)
            issue1(2 * h + 3, q1, sem_b)
            return carry

        lax.fori_loop(0, NCH // 2 - 1, pair, 0)
        drain1(q0, sem_a)
        pltpu.sync_copy(q0, acc_sh.at[didx.at[NCH - 2]], add=True)
        drain1(q1, sem_b)
        pltpu.sync_copy(q1, acc_sh.at[didx.at[NCH - 1]], add=True)

    pltpu.sync_copy(src2_hbm.at[pl.ds(sid * NCH, NCH)], sidx)
    pltpu.sync_copy(dst2_hbm.at[pl.ds(sid * NCH, NCH)], didx)
    pl.when(cid == 0)(lambda: run(blo_hbm))
    pl.when(cid == 1)(lambda: run(bhi_hbm))

    plsc.subcore_barrier()
    for hh in range(2):
        sl = pl.ds(sid * ROWS_PW + hh * (ROWS_PW // 2), ROWS_PW // 2)
        pltpu.sync_copy(acc_sh.at[sl], stage)
        osl = pl.ds(roff + sid * ROWS_PW + hh * (ROWS_PW // 2), ROWS_PW // 2)
        pltpu.sync_copy(stage, out_hbm.at[osl])


def _segsum(src2, dst2, b_lo, b_hi, zseg):
    mesh = plsc.VectorSubcoreMesh(core_axis_name="c", subcore_axis_name="s",
                                  num_cores=NC, num_subcores=NS)
    f = pl.kernel(
        _seg_body, mesh=mesh,
        compiler_params=pltpu.CompilerParams(use_tc_tiling_on_sc=False),
        out_type=jax.ShapeDtypeStruct((2 * NPAD, DH), _F32),
        scratch_types=[
            pltpu.VMEM((NCH, ECH), jnp.int32),
            pltpu.VMEM((NCH, ECH), jnp.int32),
            pltpu.VMEM((ECH, DH), _F32),
            pltpu.VMEM((ECH, DH), _F32),
            pltpu.VMEM((ECH, DH), _F32),
            pltpu.VMEM((ECH, DH), _F32),
            pltpu.VMEM((ROWS_PW // 2, DH), _F32),
            pltpu.VMEM_SHARED((NPAD, DH), _F32),
            pltpu.SemaphoreType.DMA, pltpu.SemaphoreType.DMA,
        ],
    )
    return f(src2, dst2, b_lo, b_hi, zseg)


def _l0_body(vf_ref, pos_ref, proj_ref, w0f, w0p, w0c, w1f, w1p, w1c,
             a_ref, b_ref):
    vf = vf_ref[...]
    pos = pos_ref[...]
    proj = (proj_ref[0, :, :] + proj_ref[1, :, :]
            + proj_ref[2, :, :] + proj_ref[3, :, :])
    a_ref[...] = _dot(vf, w0f[...]) + _dot(pos, w0p[...]) + _dot(proj, w0c[...])
    b = _dot(vf, w1f[...]) + _dot(pos, w1p[...]) + _dot(proj, w1c[...])
    b_ref[0, :, :] = b[:, 0:DH]
    b_ref[1, :, :] = b[:, DH:D]


def _layer0(vfeat, pos8, proj, w0f, w0p, w0c, w1f, w1p, w1c, bm=2048):
    grid = (NPAD // bm,)
    wspec = lambda shp: pl.BlockSpec(shp, lambda i: (0, 0))
    return pl.pallas_call(
        _l0_body,
        grid=grid,
        in_specs=[
            pl.BlockSpec((bm, D), lambda i: (i, 0)),
            pl.BlockSpec((bm, 8), lambda i: (i, 0)),
            pl.BlockSpec((4, bm, D), lambda i: (0, i, 0)),
            wspec((D, D)), wspec((8, D)), wspec((D, D)),
            wspec((D, D)), wspec((8, D)), wspec((D, D)),
        ],
        out_specs=(pl.BlockSpec((bm, D), lambda i: (i, 0)),
                   pl.BlockSpec((2, bm, DH), lambda i: (0, i, 0))),
        out_shape=(jax.ShapeDtypeStruct((NPAD, D), _F32),
                   jax.ShapeDtypeStruct((2, NPAD, DH), _F32)),
    )(vfeat, pos8, proj, w0f, w0p, w0c, w1f, w1p, w1c)


def _lk_body(aprev_ref, plo_ref, phi_ref, pos_ref, w0f, w0p, w1f, w1p,
             a_ref, b_ref):
    ap = aprev_ref[...]
    nfl = jnp.maximum(ap[:, 0:DH] + plo_ref[...], 0.0)
    nfh = jnp.maximum(ap[:, DH:D] + phi_ref[...], 0.0)
    pos = pos_ref[...]
    a_ref[...] = (_dot(nfl, w0f[0:DH, :]) + _dot(nfh, w0f[DH:D, :])
                  + _dot(pos, w0p[...]))
    b = (_dot(nfl, w1f[0:DH, :]) + _dot(nfh, w1f[DH:D, :])
         + _dot(pos, w1p[...]))
    b_ref[0, :, :] = b[:, 0:DH]
    b_ref[1, :, :] = b[:, DH:D]


def _layerk(aprev, parts, pos8, w0f, w0p, w1f, w1p, bm=2048):
    grid = (NPAD // bm,)
    nb = NPAD // bm
    wspec = lambda shp: pl.BlockSpec(shp, lambda i: (0, 0))
    return pl.pallas_call(
        _lk_body,
        grid=grid,
        in_specs=[
            pl.BlockSpec((bm, D), lambda i: (i, 0)),
            pl.BlockSpec((bm, DH), lambda i: (i, 0)),
            pl.BlockSpec((bm, DH), lambda i: (i + nb, 0)),
            pl.BlockSpec((bm, 8), lambda i: (i, 0)),
            wspec((D, D)), wspec((8, D)), wspec((D, D)), wspec((8, D)),
        ],
        out_specs=(pl.BlockSpec((bm, D), lambda i: (i, 0)),
                   pl.BlockSpec((2, bm, DH), lambda i: (0, i, 0))),
        out_shape=(jax.ShapeDtypeStruct((NPAD, D), _F32),
                   jax.ShapeDtypeStruct((2, NPAD, DH), _F32)),
    )(aprev, parts, parts, pos8, w0f, w0p, w1f, w1p)


def _fin_body(aprev_ref, plo_ref, phi_ref, pos_ref, wl1t, nf_ref, np_ref):
    ap = aprev_ref[...]
    nfl = jnp.maximum(ap[:, 0:DH] + plo_ref[...], 0.0)
    nfh = jnp.maximum(ap[:, DH:D] + phi_ref[...], 0.0)
    nf_ref[:, 0:DH] = nfl
    nf_ref[:, DH:D] = nfh
    np_ref[...] = pos_ref[...] + jnp.tanh(
        _dot(nfl, wl1t[0:DH, :]) + _dot(nfh, wl1t[DH:D, :]))


def _final(aprev, parts, pos8, wl1t, bm=2048):
    grid = (NPAD // bm,)
    nb = NPAD // bm
    wspec = lambda shp: pl.BlockSpec(shp, lambda i: (0, 0))
    return pl.pallas_call(
        _fin_body,
        grid=grid,
        in_specs=[
            pl.BlockSpec((bm, D), lambda i: (i, 0)),
            pl.BlockSpec((bm, DH), lambda i: (i, 0)),
            pl.BlockSpec((bm, DH), lambda i: (i + nb, 0)),
            pl.BlockSpec((bm, 8), lambda i: (i, 0)),
            wspec((D, 8)),
        ],
        out_specs=(pl.BlockSpec((bm, D), lambda i: (i, 0)),
                   pl.BlockSpec((bm, 8), lambda i: (i, 0))),
        out_shape=(jax.ShapeDtypeStruct((NPAD, D), _F32),
                   jax.ShapeDtypeStruct((NPAD, 8), _F32)),
    )(aprev, parts, parts, pos8, wl1t)


def kernel(vertex_positions, vertex_features, edge_index, feat0, feat1,
           feat2, feat3, W_lin0, w0_g0, w1_g0, w0_g1, w1_g1, w0_g2, w1_g2,
           W_lin1):
    f32 = _F32
    npadv = NPAD - N
    pos8 = jnp.pad(vertex_positions, ((0, npadv), (0, 5)))
    vfeat = jnp.pad(vertex_features, ((0, npadv), (0, 0)))
    fts = []
    for f, c, sz, pp in zip((feat0, feat1, feat2, feat3), CHANS, SIZES, PPAD):
        ft = f.reshape(c, sz * sz).T
        fts.append(jnp.pad(ft, ((0, pp - sz * sz), (0, 0))))
    wt = W_lin0.T

    def wpad3(wm):
        wp = jnp.pad(wm[0:3], ((0, 5), (0, 0)))
        return wm[3:], wp

    def wsplit0(wm):
        wp = jnp.pad(wm[D:D + 3], ((0, 5), (0, 0)))
        return wm[0:D], wp, wm[D + 3:]

    w0f, w0p, w0c = wsplit0(w0_g0)
    w1f, w1p, w1c = wsplit0(w1_g0)
    w0f1, w0p1 = wpad3(w0_g1)
    w1f1, w1p1 = wpad3(w1_g1)
    w0f2, w0p2 = wpad3(w0_g2)
    w1f2, w1p2 = wpad3(w1_g2)
    wl1t = jnp.pad(W_lin1.T, ((0, 0), (0, 5)))
    npade = EPAD - E
    src2 = jnp.concatenate(
        [edge_index[0], jnp.zeros((npade,), jnp.int32)]).reshape(-1, ECH)
    dst2 = jnp.concatenate(
        [edge_index[1], jnp.full((npade,), NPAD - 1, jnp.int32)]
    ).reshape(-1, ECH)
    zseg = jnp.zeros((ROWS_PW // 2, DH), f32)

    idx = _gather_indices(vertex_positions)

    t = _prep(*fts, wt)
    proj = _proj(t, idx.reshape(-1, VCH))
    a0, b0 = _layer0(vfeat, pos8, proj, w0f, w0p, w0c, w1f, w1p, w1c)
    parts = _segsum(src2, dst2, b0[0], b0[1], zseg)
    a1, b1 = _layerk(a0, parts, pos8, w0f1, w0p1, w1f1, w1p1)
    parts = _segsum(src2, dst2, b1[0], b1[1], zseg)
    a2, b2 = _layerk(a1, parts, pos8, w0f2, w0p2, w1f2, w1p2)
    parts = _segsum(src2, dst2, b2[0], b2[1], zseg)
    nf, npos = _final(a2, parts, pos8, wl1t)
    return npos[:N, :3], nf[:N, :]


# speedup vs baseline: 6.3140x; 2.1528x over previous
import functools

import jax
import jax.numpy as jnp
from jax import lax
from jax.experimental import pallas as pl
from jax.experimental.pallas import tpu as pltpu
from jax.experimental.pallas import tpu_sc as plsc

N = 10000
NPAD = 10240
E = 320000
D = 128
SIZES = (56, 28, 14, 7)
CHANS = (256, 512, 1024, 2048)
COFF = (0, 256, 768, 1792, 3840)
PPAD = (3136, 784, 200, 56)
OFFS = (0, 3136, 3920, 4120)
ZROW = 4176
TROWS = 4184

NC, NS = 2, 16
NW = NC * NS
VPW = NPAD // NW
VCH = 80
ECH = 128
NCH = 160
EPT = NCH * ECH
EPAD = NS * EPT
ROWS_PW = NPAD // NS
DH = D // 2
ACC_P = NS * VPW

_PREC = None
_F32 = jnp.float32


def _dot(a, b):
    return jnp.dot(a, b, preferred_element_type=_F32, precision=_PREC)


def _prep_body(f0_ref, f1_ref, f2_ref, f3_ref, wt_ref, t_ref):
    t_ref[0:3136, :] = _dot(f0_ref[...], wt_ref[COFF[0]:COFF[1], :])
    t_ref[3136:3920, :] = _dot(f1_ref[...], wt_ref[COFF[1]:COFF[2], :])
    t_ref[3920:4120, :] = _dot(f2_ref[...], wt_ref[COFF[2]:COFF[3], :])
    t_ref[4120:4176, :] = _dot(f3_ref[...], wt_ref[COFF[3]:COFF[4], :])
    t_ref[4176:4184, :] = jnp.zeros((8, D), _F32)


def _prep(f0t, f1t, f2t, f3t, wt):
    return pl.pallas_call(
        _prep_body,
        out_shape=jax.ShapeDtypeStruct((TROWS, D), _F32),
    )(f0t, f1t, f2t, f3t, wt)


def _gather_indices(vertex_positions):
    z = vertex_positions[:, 2]
    h = 248.0 * (vertex_positions[:, 1] / z) + 111.5
    w = 248.0 * (vertex_positions[:, 0] / (-z)) + 111.5
    h = jnp.clip(h, 0.0, 223.0)
    w = jnp.clip(w, 0.0, 223.0)
    gcols, mcols, lcols = [], [], []
    for s in range(4):
        size = SIZES[s]
        x = w / (224.0 / size)
        y = h / (224.0 / size)
        x1 = jnp.floor(x).astype(jnp.int32)
        y1 = jnp.floor(y).astype(jnp.int32)
        x2 = jnp.minimum(jnp.ceil(x).astype(jnp.int32), size - 1)
        y2 = jnp.minimum(jnp.ceil(y).astype(jnp.int32), size - 1)
        xi = x.astype(jnp.int32)
        yi = y.astype(jnp.int32)
        m = (x2 - xi) * (y2 - yi)
        lin = jnp.clip(x1 * size + y1, 0, size * size - 1)
        if s < 2:
            gcols.append(OFFS[s] + lin)
            mcols.append(m.astype(jnp.float32))
        else:
            lcols.append(jnp.where(m == 1, lin, PPAD[s]))
    idx01 = jnp.pad(jnp.stack(gcols), ((0, 0), (0, NPAD - N)))
    m4 = jnp.pad(jnp.stack(mcols, axis=1), ((0, NPAD - N), (0, 6)))
    lin23 = jnp.pad(jnp.stack(lcols, axis=1), ((0, NPAD - N), (0, 6)),
                    constant_values=PPAD[3])
    return idx01, m4, lin23


def _proj_body(t_hbm, idx2_hbm, out_hbm,
               idxp, r0, r1, r2, r3, sem_a, sem_b):
    cid = lax.axis_index("c")
    sid = lax.axis_index("s")
    wid = cid * NS + sid
    base = wid * VPW
    set_a = (r0, r1)
    set_b = (r2, r3)

    for s in range(2):
        pltpu.sync_copy(idx2_hbm.at[pl.ds(s * 128 + wid * 4, 4)],
                        idxp.at[pl.ds(s * 4, 4)])

    def issue(k, st, sem):
        for s in range(2):
            pltpu.async_copy(t_hbm.at[idxp.at[s * 4 + k]], st[s], sem)

    def drain(st, sem):
        for s in range(2):
            pltpu.make_async_copy(t_hbm.at[pl.ds(0, VCH)], st[s], sem).wait()

    def write_out(k, st):
        for s in range(2):
            pltpu.sync_copy(st[s],
                            out_hbm.at[s, pl.ds(base + k * VCH, VCH)])

    issue(0, set_a, sem_a)
    issue(1, set_b, sem_b)
    drain(set_a, sem_a)
    write_out(0, set_a)
    issue(2, set_a, sem_a)
    drain(set_b, sem_b)
    write_out(1, set_b)
    issue(3, set_b, sem_b)
    drain(set_a, sem_a)
    write_out(2, set_a)
    drain(set_b, sem_b)
    write_out(3, set_b)


def _proj(t, idx2):
    mesh = plsc.VectorSubcoreMesh(core_axis_name="c", subcore_axis_name="s",
                                  num_cores=NC, num_subcores=NS)
    f = pl.kernel(
        _proj_body, mesh=mesh,
        compiler_params=pltpu.CompilerParams(use_tc_tiling_on_sc=False),
        out_type=jax.ShapeDtypeStruct((2, NPAD, D), _F32),
        scratch_types=[pltpu.VMEM((8, VCH), jnp.int32)]
        + [pltpu.VMEM((VCH, D), _F32)] * 4
        + [pltpu.SemaphoreType.DMA, pltpu.SemaphoreType.DMA],
    )
    return f(t, idx2)


def _seg_body(src2_hbm, dst2_hbm, blo_hbm, bhi_hbm, z_hbm, out_hbm,
              sidx, didx, q0, q1, q2, q3, stage, acc_sh, sem_a, sem_b):
    cid = lax.axis_index("c")
    sid = lax.axis_index("s")
    roff = cid * NPAD
    rows = q0

    pltpu.sync_copy(z_hbm, stage)
    pltpu.sync_copy(stage, acc_sh.at[pl.ds(sid * ROWS_PW, ROWS_PW // 2)])
    pltpu.sync_copy(stage,
                    acc_sh.at[pl.ds(sid * ROWS_PW + ROWS_PW // 2, ROWS_PW // 2)])
    plsc.subcore_barrier()

    def run(b_ref):
        def issue1(ch, buf, sem):
            pltpu.async_copy(b_ref.at[sidx.at[ch]], buf, sem)

        def drain1(buf, sem):
            pltpu.make_async_copy(b_ref.at[pl.ds(0, ECH)], buf, sem).wait()

        issue1(0, q0, sem_a)
        issue1(1, q1, sem_b)

        def pair(h, carry):
            drain1(q0, sem_a)
            pltpu.sync_copy(q0, acc_sh.at[didx.at[2 * h]], add=True)
            issue1(2 * h + 2, q0, sem_a)
            drain1(q1, sem_b)
            pltpu.sync_copy(q1, acc_sh.at[didx.at[2 * h + 1]], add=True)
            issue1(2 * h + 3, q1, sem_b)
            return carry

        lax.fori_loop(0, NCH // 2 - 1, pair, 0)
        drain1(q0, sem_a)
        pltpu.sync_copy(q0, acc_sh.at[didx.at[NCH - 2]], add=True)
        drain1(q1, sem_b)
        pltpu.sync_copy(q1, acc_sh.at[didx.at[NCH - 1]], add=True)

    pltpu.sync_copy(src2_hbm.at[pl.ds(sid * NCH, NCH)], sidx)
    pltpu.sync_copy(dst2_hbm.at[pl.ds(sid * NCH, NCH)], didx)
    pl.when(cid == 0)(lambda: run(blo_hbm))
    pl.when(cid == 1)(lambda: run(bhi_hbm))

    plsc.subcore_barrier()
    for hh in range(2):
        sl = pl.ds(sid * ROWS_PW + hh * (ROWS_PW // 2), ROWS_PW // 2)
        pltpu.sync_copy(acc_sh.at[sl], stage)
        osl = pl.ds(roff + sid * ROWS_PW + hh * (ROWS_PW // 2), ROWS_PW // 2)
        pltpu.sync_copy(stage, out_hbm.at[osl])


def _segsum(src2, dst2, b_lo, b_hi, zseg):
    mesh = plsc.VectorSubcoreMesh(core_axis_name="c", subcore_axis_name="s",
                                  num_cores=NC, num_subcores=NS)
    f = pl.kernel(
        _seg_body, mesh=mesh,
        compiler_params=pltpu.CompilerParams(use_tc_tiling_on_sc=False),
        out_type=jax.ShapeDtypeStruct((2 * NPAD, DH), _F32),
        scratch_types=[
            pltpu.VMEM((NCH, ECH), jnp.int32),
            pltpu.VMEM((NCH, ECH), jnp.int32),
            pltpu.VMEM((ECH, DH), _F32),
            pltpu.VMEM((ECH, DH), _F32),
            pltpu.VMEM((ECH, DH), _F32),
            pltpu.VMEM((ECH, DH), _F32),
            pltpu.VMEM((ROWS_PW // 2, DH), _F32),
            pltpu.VMEM_SHARED((NPAD, DH), _F32),
            pltpu.SemaphoreType.DMA, pltpu.SemaphoreType.DMA,
        ],
    )
    return f(src2, dst2, b_lo, b_hi, zseg)


def _l0_body(vf_ref, pos_ref, proj_ref, m_ref, lin_ref, t2_ref, t3_ref,
             w0f, w0p, w0c, w1f, w1p, w1c, a_ref, b_ref):
    vf = vf_ref[...]
    pos = pos_ref[...]
    mm = m_ref[...]
    lin = lin_ref[...]
    bm = vf.shape[0]
    proj = (mm[:, 0:1] * proj_ref[0, :, :] + mm[:, 1:2] * proj_ref[1, :, :])
    oh2 = (lax.broadcasted_iota(jnp.int32, (bm, PPAD[2]), 1)
           == lin[:, 0:1]).astype(_F32)
    oh3 = (lax.broadcasted_iota(jnp.int32, (bm, PPAD[3]), 1)
           == lin[:, 1:2]).astype(_F32)
    proj = proj + _dot(oh2, t2_ref[...]) + _dot(oh3, t3_ref[...])
    a_ref[...] = _dot(vf, w0f[...]) + _dot(pos, w0p[...]) + _dot(proj, w0c[...])
    b = _dot(vf, w1f[...]) + _dot(pos, w1p[...]) + _dot(proj, w1c[...])
    b_ref[0, :, :] = b[:, 0:DH]
    b_ref[1, :, :] = b[:, DH:D]


def _layer0(vfeat, pos8, proj, m4, lin23, t2, t3,
            w0f, w0p, w0c, w1f, w1p, w1c, bm=2048):
    grid = (NPAD // bm,)
    wspec = lambda shp: pl.BlockSpec(shp, lambda i: (0, 0))
    return pl.pallas_call(
        _l0_body,
        grid=grid,
        in_specs=[
            pl.BlockSpec((bm, D), lambda i: (i, 0)),
            pl.BlockSpec((bm, 8), lambda i: (i, 0)),
            pl.BlockSpec((2, bm, D), lambda i: (0, i, 0)),
            pl.BlockSpec((bm, 8), lambda i: (i, 0)),
            pl.BlockSpec((bm, 8), lambda i: (i, 0)),
            wspec((PPAD[2], D)), wspec((PPAD[3], D)),
            wspec((D, D)), wspec((8, D)), wspec((D, D)),
            wspec((D, D)), wspec((8, D)), wspec((D, D)),
        ],
        out_specs=(pl.BlockSpec((bm, D), lambda i: (i, 0)),
                   pl.BlockSpec((2, bm, DH), lambda i: (0, i, 0))),
        out_shape=(jax.ShapeDtypeStruct((NPAD, D), _F32),
                   jax.ShapeDtypeStruct((2, NPAD, DH), _F32)),
    )(vfeat, pos8, proj, m4, lin23, t2, t3, w0f, w0p, w0c, w1f, w1p, w1c)


def _lk_body(aprev_ref, plo_ref, phi_ref, pos_ref, w0f, w0p, w1f, w1p,
             a_ref, b_ref):
    ap = aprev_ref[...]
    nfl = jnp.maximum(ap[:, 0:DH] + plo_ref[...], 0.0)
    nfh = jnp.maximum(ap[:, DH:D] + phi_ref[...], 0.0)
    pos = pos_ref[...]
    a_ref[...] = (_dot(nfl, w0f[0:DH, :]) + _dot(nfh, w0f[DH:D, :])
                  + _dot(pos, w0p[...]))
    b = (_dot(nfl, w1f[0:DH, :]) + _dot(nfh, w1f[DH:D, :])
         + _dot(pos, w1p[...]))
    b_ref[0, :, :] = b[:, 0:DH]
    b_ref[1, :, :] = b[:, DH:D]


def _layerk(aprev, parts, pos8, w0f, w0p, w1f, w1p, bm=2048):
    grid = (NPAD // bm,)
    nb = NPAD // bm
    wspec = lambda shp: pl.BlockSpec(shp, lambda i: (0, 0))
    return pl.pallas_call(
        _lk_body,
        grid=grid,
        in_specs=[
            pl.BlockSpec((bm, D), lambda i: (i, 0)),
            pl.BlockSpec((bm, DH), lambda i: (i, 0)),
            pl.BlockSpec((bm, DH), lambda i: (i + nb, 0)),
            pl.BlockSpec((bm, 8), lambda i: (i, 0)),
            wspec((D, D)), wspec((8, D)), wspec((D, D)), wspec((8, D)),
        ],
        out_specs=(pl.BlockSpec((bm, D), lambda i: (i, 0)),
                   pl.BlockSpec((2, bm, DH), lambda i: (0, i, 0))),
        out_shape=(jax.ShapeDtypeStruct((NPAD, D), _F32),
                   jax.ShapeDtypeStruct((2, NPAD, DH), _F32)),
    )(aprev, parts, parts, pos8, w0f, w0p, w1f, w1p)


def _fin_body(aprev_ref, plo_ref, phi_ref, pos_ref, wl1t, nf_ref, np_ref):
    ap = aprev_ref[...]
    nfl = jnp.maximum(ap[:, 0:DH] + plo_ref[...], 0.0)
    nfh = jnp.maximum(ap[:, DH:D] + phi_ref[...], 0.0)
    nf_ref[:, 0:DH] = nfl
    nf_ref[:, DH:D] = nfh
    np_ref[...] = pos_ref[...] + jnp.tanh(
        _dot(nfl, wl1t[0:DH, :]) + _dot(nfh, wl1t[DH:D, :]))


def _final(aprev, parts, pos8, wl1t, bm=2048):
    grid = (NPAD // bm,)
    nb = NPAD // bm
    wspec = lambda shp: pl.BlockSpec(shp, lambda i: (0, 0))
    return pl.pallas_call(
        _fin_body,
        grid=grid,
        in_specs=[
            pl.BlockSpec((bm, D), lambda i: (i, 0)),
            pl.BlockSpec((bm, DH), lambda i: (i, 0)),
            pl.BlockSpec((bm, DH), lambda i: (i + nb, 0)),
            pl.BlockSpec((bm, 8), lambda i: (i, 0)),
            wspec((D, 8)),
        ],
        out_specs=(pl.BlockSpec((bm, D), lambda i: (i, 0)),
                   pl.BlockSpec((bm, 8), lambda i: (i, 0))),
        out_shape=(jax.ShapeDtypeStruct((NPAD, D), _F32),
                   jax.ShapeDtypeStruct((NPAD, 8), _F32)),
    )(aprev, parts, parts, pos8, wl1t)


def kernel(vertex_positions, vertex_features, edge_index, feat0, feat1,
           feat2, feat3, W_lin0, w0_g0, w1_g0, w0_g1, w1_g1, w0_g2, w1_g2,
           W_lin1):
    f32 = _F32
    npadv = NPAD - N
    pos8 = jnp.pad(vertex_positions, ((0, npadv), (0, 5)))
    vfeat = jnp.pad(vertex_features, ((0, npadv), (0, 0)))
    fts = []
    for f, c, sz, pp in zip((feat0, feat1, feat2, feat3), CHANS, SIZES, PPAD):
        ft = f.reshape(c, sz * sz).T
        fts.append(jnp.pad(ft, ((0, pp - sz * sz), (0, 0))))
    wt = W_lin0.T

    def wpad3(wm):
        wp = jnp.pad(wm[0:3], ((0, 5), (0, 0)))
        return wm[3:], wp

    def wsplit0(wm):
        wp = jnp.pad(wm[D:D + 3], ((0, 5), (0, 0)))
        return wm[0:D], wp, wm[D + 3:]

    w0f, w0p, w0c = wsplit0(w0_g0)
    w1f, w1p, w1c = wsplit0(w1_g0)
    w0f1, w0p1 = wpad3(w0_g1)
    w1f1, w1p1 = wpad3(w1_g1)
    w0f2, w0p2 = wpad3(w0_g2)
    w1f2, w1p2 = wpad3(w1_g2)
    wl1t = jnp.pad(W_lin1.T, ((0, 0), (0, 5)))
    npade = EPAD - E
    src2 = jnp.concatenate(
        [edge_index[0], jnp.zeros((npade,), jnp.int32)]).reshape(-1, ECH)
    dst2 = jnp.concatenate(
        [edge_index[1], jnp.full((npade,), NPAD - 1, jnp.int32)]
    ).reshape(-1, ECH)
    zseg = jnp.zeros((ROWS_PW // 2, DH), f32)

    idx01, m4, lin23 = _gather_indices(vertex_positions)

    t = _prep(*fts, wt)
    proj = _proj(t, idx01.reshape(-1, VCH))
    a0, b0 = _layer0(vfeat, pos8, proj, m4, lin23,
                     t[OFFS[2]:OFFS[3]], t[OFFS[3]:OFFS[3] + PPAD[3]],
                     w0f, w0p, w0c, w1f, w1p, w1c)
    parts = _segsum(src2, dst2, b0[0], b0[1], zseg)
    a1, b1 = _layerk(a0, parts, pos8, w0f1, w0p1, w1f1, w1p1)
    parts = _segsum(src2, dst2, b1[0], b1[1], zseg)
    a2, b2 = _layerk(a1, parts, pos8, w0f2, w0p2, w1f2, w1p2)
    parts = _segsum(src2, dst2, b2[0], b2[1], zseg)
    nf, npos = _final(a2, parts, pos8, wl1t)
    return npos[:N, :3], nf[:N, :]
